# baseline (speedup 1.0000x reference)
import dataclasses
import functools

import jax
import jax.numpy as jnp
from jax import lax
from jax.experimental import pallas as pl
from jax.experimental.pallas import tpu as pltpu
from jax.experimental.pallas import tpu_sc as plsc

D = 128
HALF = 64
DG = 80
NC, NS, LN = 2, 16, 16
CHUNK = 48
NBLK = 2000


def _ceil_div(a, b):
    return (a + b - 1) // b


def _mm_body(x_ref, w_ref, as_ref, ad_ref, he_ref, sd_ref):
    i = pl.program_id(0)
    h = jnp.dot(x_ref[...], w_ref[...], preferred_element_type=jnp.float32)
    lane = lax.broadcasted_iota(jnp.int32, (h.shape[0], DG - HALF), 1)
    marker = jnp.where(lane == 0, 1.0, 0.0)
    he_ref[0] = jnp.concatenate([h[:, :HALF], marker], axis=1)
    he_ref[1] = jnp.concatenate([h[:, HALF:], marker], axis=1)
    s = jnp.dot(h, as_ref[0, :], preferred_element_type=jnp.float32)
    d = jnp.dot(h, ad_ref[0, :], preferred_element_type=jnp.float32)
    sd_ref[0] = jnp.stack([s, d])


def _mm_norm_body(x_ref, mustd_ref, w_ref, as_ref, ad_ref, he_ref, sd_ref):
    i = pl.program_id(0)
    xn = (x_ref[...] - mustd_ref[0, :]) * mustd_ref[1, :]
    xn = jnp.maximum(xn, 0.0)
    h = jnp.dot(xn, w_ref[...], preferred_element_type=jnp.float32)
    lane = lax.broadcasted_iota(jnp.int32, (h.shape[0], DG - HALF), 1)
    marker = jnp.where(lane == 0, 1.0, 0.0)
    he_ref[0] = jnp.concatenate([h[:, :HALF], marker], axis=1)
    he_ref[1] = jnp.concatenate([h[:, HALF:], marker], axis=1)
    s = jnp.dot(h, as_ref[0, :], preferred_element_type=jnp.float32)
    d = jnp.dot(h, ad_ref[0, :], preferred_element_type=jnp.float32)
    sd_ref[0] = jnp.stack([s, d])


def _project(x, W, a_s, a_d, mustd=None):
    n = x.shape[0]
    grid = (n // NBLK,)
    in_specs = [pl.BlockSpec((NBLK, D), lambda i: (i, 0))]
    args = [x]
    body = _mm_body
    if mustd is not None:
        in_specs.append(pl.BlockSpec((2, D), lambda i: (0, 0)))
        args.append(mustd)
        body = _mm_norm_body
    in_specs += [
        pl.BlockSpec((D, D), lambda i: (0, 0)),
        pl.BlockSpec((1, D), lambda i: (0, 0)),
        pl.BlockSpec((1, D), lambda i: (0, 0)),
    ]
    args += [W, a_s.reshape(1, D), a_d.reshape(1, D)]
    return pl.pallas_call(
        body,
        grid=grid,
        in_specs=in_specs,
        out_specs=[
            pl.BlockSpec((2, NBLK, DG), lambda i: (0, i, 0)),
            pl.BlockSpec((1, 2, NBLK), lambda i: (i, 0, 0)),
        ],
        out_shape=[
            jax.ShapeDtypeStruct((2, n, DG), jnp.float32),
            jax.ShapeDtypeStruct((n // NBLK, 2, NBLK), jnp.float32),
        ],
    )(*args)


def _et_body(ea_ref, ec_ref, et_ref, sum_ref, acc):
    i = pl.program_id(0)
    v = jnp.dot(ea_ref[...], ec_ref[0, :], preferred_element_type=jnp.float32)
    et_ref[0, :] = v

    @pl.when(i == 0)
    def _():
        acc[...] = jnp.zeros_like(acc)

    acc[0, :] += jnp.sum(v.reshape(-1, D), axis=0)
    sum_ref[...] = acc[...]


def _edge_terms(edge_att, ec):
    e = edge_att.shape[0]
    blk = 12800
    return pl.pallas_call(
        _et_body,
        grid=(e // blk,),
        in_specs=[
            pl.BlockSpec((blk, ec.shape[0] if False else 16), lambda i: (i, 0)),
            pl.BlockSpec((1, 16), lambda i: (0, 0)),
        ],
        out_specs=[
            pl.BlockSpec((1, blk), lambda i: (0, i)),
            pl.BlockSpec((1, D), lambda i: (0, 0)),
        ],
        out_shape=[
            jax.ShapeDtypeStruct((1, e), jnp.float32),
            jax.ShapeDtypeStruct((1, D), jnp.float32),
        ],
        scratch_shapes=[pltpu.VMEM((1, D), jnp.float32)],
    )(edge_att, ec.reshape(1, 16))


def _merge_body(a0_ref, a1_ref, b_ref, out_ref, stats_ref, acc):
    i = pl.program_id(0)
    a0 = a0_ref[...]
    a1 = a1_ref[...]
    num = jnp.concatenate([a0[:, :HALF], a1[:, :HALF]], axis=1)
    den = a0[:, HALF]
    out = num / (den[:, None] + 1e-16) + b_ref[0, :]
    out_ref[...] = out

    @pl.when(i == 0)
    def _():
        acc[...] = jnp.zeros_like(acc)

    acc[0, :] += jnp.sum(out, axis=0)
    acc[1, :] += jnp.sum(out * out, axis=0)
    stats_ref[...] = acc[...]


def _merge(acc, b, n):
    nb = n // NBLK
    return pl.pallas_call(
        _merge_body,
        grid=(nb,),
        in_specs=[
            pl.BlockSpec((NBLK, DG), lambda i: (i, 0)),
            pl.BlockSpec((NBLK, DG), lambda i: (i + nb, 0)),
            pl.BlockSpec((1, D), lambda i: (0, 0)),
        ],
        out_specs=[
            pl.BlockSpec((NBLK, D), lambda i: (i, 0)),
            pl.BlockSpec((2, D), lambda i: (0, 0)),
        ],
        out_shape=[
            jax.ShapeDtypeStruct((n, D), jnp.float32),
            jax.ShapeDtypeStruct((2, D), jnp.float32),
        ],
        scratch_shapes=[pltpu.VMEM((2, D), jnp.float32)],
    )(acc, acc, b.reshape(1, D))


def _norm_body(x_ref, mustd_ref, o_ref):
    o_ref[...] = (x_ref[...] - mustd_ref[0, :]) * mustd_ref[1, :]


def _normalize(x, mustd):
    n = x.shape[0]
    return pl.pallas_call(
        _norm_body,
        grid=(n // NBLK,),
        in_specs=[
            pl.BlockSpec((NBLK, D), lambda i: (i, 0)),
            pl.BlockSpec((2, D), lambda i: (0, 0)),
        ],
        out_specs=pl.BlockSpec((NBLK, D), lambda i: (i, 0)),
        out_shape=jax.ShapeDtypeStruct((n, D), jnp.float32),
    )(x, mustd)


def _sc_edge_pass(h_ext, sd, src2, dst2, et2, n, nch):
    ngrp = nch // 4
    rsub = (n // NS) // 8 * 8
    rlast = n - (NS - 1) * rsub
    nz = rsub // 48

    mesh = plsc.VectorSubcoreMesh(
        core_axis_name="c", subcore_axis_name="s", num_cores=NC)
    cp = pltpu.CompilerParams()
    for fld, val in (("needs_layout_passes", False),
                     ("use_tc_tiling_on_sc", False)):
        if fld in pltpu.CompilerParams.__dataclass_fields__:
            cp = dataclasses.replace(cp, **{fld: val})

    @functools.partial(
        pl.kernel,
        out_type=jax.ShapeDtypeStruct((NC * n, DG), jnp.float32),
        mesh=mesh,
        compiler_params=cp,
        scratch_types=[
            pltpu.VMEM((2, n), jnp.float32),
            pltpu.VMEM((2, CHUNK), jnp.int32),
            pltpu.VMEM((2, CHUNK), jnp.int32),
            pltpu.VMEM((2, CHUNK), jnp.float32),
            pltpu.VMEM((2, CHUNK), jnp.int32),
            pltpu.VMEM((2, CHUNK), jnp.int32),
            pltpu.VMEM((2, CHUNK), jnp.float32),
            pltpu.VMEM((CHUNK, DG), jnp.float32),
            pltpu.VMEM((CHUNK, DG), jnp.float32),
            pltpu.SemaphoreType.DMA,
            pltpu.SemaphoreType.DMA,
            pltpu.SemaphoreType.DMA,
            pltpu.SemaphoreType.DMA,
            pltpu.SemaphoreType.DMA,
            pltpu.SemaphoreType.DMA,
            pltpu.VMEM_SHARED((n, DG), jnp.float32),
        ],
    )
    def k(h_hbm, sd_hbm, src_hbm, dst_hbm, et_hbm, out_hbm,
          sd_v, srcA, dstA, etA, srcB, dstB, etB, rowsA, rowsB,
          gA, gB, sA, sB, wA, wB, acc_sh):
        sid = lax.axis_index("s")
        cid = lax.axis_index("c")
        off = cid * n
        offv = jnp.full((LN,), off, jnp.int32)

        zero16 = jnp.zeros((LN,), jnp.float32)

        @pl.loop(0, 48)
        def _(r):
            for cc in range(DG // LN):
                rowsA[r, pl.ds(cc * LN, LN)] = zero16

        pltpu.sync_copy(sd_hbm, sd_v)

        base = sid * rsub

        @pl.loop(0, nz)
        def _(j):
            pltpu.sync_copy(rowsA.at[pl.ds(0, 48)],
                            acc_sh.at[pl.ds(base + j * 48, 48)])

        @pl.when(sid == NS - 1)
        def _():
            pltpu.sync_copy(rowsA.at[pl.ds(0, 16)],
                            acc_sh.at[pl.ds(base + nz * 48, 16)])

        plsc.subcore_barrier()

        zeros_i = jnp.zeros((LN,), jnp.int32)
        ones_i = jnp.ones((LN,), jnp.int32)

        def compute(rows_v, srcw, dstw, etw, r):
            @pl.loop(0, CHUNK // LN)
            def _(g):
                isv = srcw[r, pl.ds(g * LN, LN)] - offv
                idv = dstw[r, pl.ds(g * LN, LN)]
                sg = plsc.load_gather(sd_v, [zeros_i, isv])
                dg = plsc.load_gather(sd_v, [ones_i, idv])
                a = sg + dg + etw[r, pl.ds(g * LN, LN)]
                a = jnp.maximum(a, a * 0.2)
                exv = jnp.exp(a)
                for j in range(LN):
                    w = exv[j]
                    e = g * LN + j
                    for rr in range(DG // LN):
                        slc = (e, pl.ds(rr * LN, LN))
                        rows_v[slc] = rows_v[slc] * w

        def win_fetch(p, dst3, sem):
            sw, dw, ew = dst3
            pltpu.async_copy(src_hbm.at[sid, pl.ds(p * 2, 2)], sw, sem)
            pltpu.async_copy(dst_hbm.at[sid, pl.ds(p * 2, 2)], dw, sem)
            pltpu.async_copy(et_hbm.at[sid, pl.ds(p * 2, 2)], ew, sem)

        def win_wait(p, dst3, sem):
            sw, dw, ew = dst3
            pltpu.make_async_copy(src_hbm.at[sid, pl.ds(p * 2, 2)], sw, sem).wait()
            pltpu.make_async_copy(dst_hbm.at[sid, pl.ds(p * 2, 2)], dw, sem).wait()
            pltpu.make_async_copy(et_hbm.at[sid, pl.ds(p * 2, 2)], ew, sem).wait()
            for r in range(2):
                for gg in range(CHUNK // LN):
                    sl = (r, pl.ds(gg * LN, LN))
                    sw[sl] = sw[sl] + offv

        def gather(srcw, r, rows_v, sem):
            pltpu.async_copy(h_hbm.at[srcw.at[r]], rows_v, sem)

        def gather_wait(srcw, r, rows_v, sem):
            pltpu.make_async_copy(h_hbm.at[srcw.at[r]], rows_v, sem).wait()

        def scatter(rows_v, dstw, r, sem):
            pltpu.async_copy(rows_v, acc_sh.at[dstw.at[r]], sem, add=True)

        def scatter_wait(rows_v, dstw, r, sem):
            pltpu.make_async_copy(rows_v, acc_sh.at[dstw.at[r]], sem).wait()

        winA = (srcA, dstA, etA)
        winB = (srcB, dstB, etB)

        win_fetch(0, winA, wA)
        win_wait(0, winA, wA)
        win_fetch(1, winB, wB)
        gather(srcA, 0, rowsA, gA)
        gather(srcA, 1, rowsB, gB)

        @pl.loop(0, ngrp - 1)
        def _(grp):
            gather_wait(srcA, 0, rowsA, gA)
            compute(rowsA, srcA, dstA, etA, 0)
            scatter(rowsA, dstA, 0, sA)
            gather_wait(srcA, 1, rowsB, gB)
            compute(rowsB, srcA, dstA, etA, 1)
            scatter(rowsB, dstA, 1, sB)
            win_wait(2 * grp + 1, winB, wB)
            scatter_wait(rowsA, dstA, 0, sA)
            gather(srcB, 0, rowsA, gA)
            scatter_wait(rowsB, dstA, 1, sB)
            gather(srcB, 1, rowsB, gB)
            win_fetch(2 * grp + 2, winA, wA)
            gather_wait(srcB, 0, rowsA, gA)
            compute(rowsA, srcB, dstB, etB, 0)
            scatter(rowsA, dstB, 0, sA)
            gather_wait(srcB, 1, rowsB, gB)
            compute(rowsB, srcB, dstB, etB, 1)
            scatter(rowsB, dstB, 1, sB)
            win_wait(2 * grp + 2, winA, wA)
            scatter_wait(rowsA, dstB, 0, sA)
            gather(srcA, 0, rowsA, gA)
            scatter_wait(rowsB, dstB, 1, sB)
            gather(srcA, 1, rowsB, gB)
            win_fetch(2 * grp + 3, winB, wB)

        gather_wait(srcA, 0, rowsA, gA)
        compute(rowsA, srcA, dstA, etA, 0)
        scatter(rowsA, dstA, 0, sA)
        gather_wait(srcA, 1, rowsB, gB)
        compute(rowsB, srcA, dstA, etA, 1)
        scatter(rowsB, dstA, 1, sB)
        win_wait(2 * ngrp - 1, winB, wB)
        scatter_wait(rowsA, dstA, 0, sA)
        gather(srcB, 0, rowsA, gA)
        scatter_wait(rowsB, dstA, 1, sB)
        gather(srcB, 1, rowsB, gB)
        gather_wait(srcB, 0, rowsA, gA)
        compute(rowsA, srcB, dstB, etB, 0)
        scatter(rowsA, dstB, 0, sA)
        gather_wait(srcB, 1, rowsB, gB)
        compute(rowsB, srcB, dstB, etB, 1)
        scatter(rowsB, dstB, 1, sB)
        scatter_wait(rowsA, dstB, 0, sA)
        scatter_wait(rowsB, dstB, 1, sB)

        plsc.subcore_barrier()

        pltpu.sync_copy(acc_sh.at[pl.ds(base, rsub)],
                        out_hbm.at[pl.ds(off + base, rsub)])

        @pl.when(sid == NS - 1)
        def _():
            pltpu.sync_copy(acc_sh.at[pl.ds(base + rsub, rlast - rsub)],
                            out_hbm.at[pl.ds(off + base + rsub, rlast - rsub)])

    return k(h_ext, sd, src2, dst2, et2)


def kernel(x, edge_index, edge_att, W1, We1, as1, ad1, ae1, b1,
           W2, We2, as2, ad2, ae2, b2):
    n = x.shape[0]
    e = edge_index.shape[1]
    ep_total = e + n
    nw = NS
    nch = _ceil_div(_ceil_div(ep_total, nw * CHUNK), 4) * 4
    ep = nw * CHUNK * nch
    pad = ep - ep_total

    loops = jnp.arange(n, dtype=jnp.int32)
    src = jnp.concatenate(
        [edge_index[0].astype(jnp.int32), loops, jnp.zeros((pad,), jnp.int32)])
    dst = jnp.concatenate(
        [edge_index[1].astype(jnp.int32), loops, jnp.zeros((pad,), jnp.int32)])
    src2 = src.reshape(nw, nch, CHUNK)
    dst2 = dst.reshape(nw, nch, CHUNK)

    ec1 = We1 @ ae1
    ec2 = We2 @ ae2
    et1_main, et1_sum = _edge_terms(edge_att, ec1)
    et2_main, et2_sum = _edge_terms(edge_att, ec2)

    def pack_et(et_main, et_sum):
        self_term = jnp.broadcast_to(jnp.sum(et_sum) / e, (n,))
        et = jnp.concatenate(
            [et_main[0], self_term, jnp.full((pad,), -1e30, jnp.float32)])
        return et.reshape(nw, nch, CHUNK)

    et1 = pack_et(et1_main, et1_sum)
    et2 = pack_et(et2_main, et2_sum)

    def bn_mustd(stats):
        mu = stats[0] / n
        var = stats[1] / n - mu * mu
        return jnp.stack([mu, 1.0 / jnp.sqrt(var + 1e-5)])

    h1e, sd1 = _project(x, W1, as1, ad1)
    h1e = h1e.reshape(2 * n, DG)
    sd1 = sd1.transpose(1, 0, 2).reshape(2, n)
    acc1 = _sc_edge_pass(h1e, sd1, src2, dst2, et1, n, nch)
    out1, stats1 = _merge(acc1, b1, n)

    h2e, sd2 = _project(out1, W2, as2, ad2, mustd=bn_mustd(stats1))
    h2e = h2e.reshape(2 * n, DG)
    sd2 = sd2.transpose(1, 0, 2).reshape(2, n)
    acc2 = _sc_edge_pass(h2e, sd2, src2, dst2, et2, n, nch)
    out2, stats2 = _merge(acc2, b2, n)

    return _normalize(out2, bn_mustd(stats2))


# speedup vs baseline: 1.0295x; 1.0295x over previous
import dataclasses
import functools

import jax
import jax.numpy as jnp
from jax import lax
from jax.experimental import pallas as pl
from jax.experimental.pallas import tpu as pltpu
from jax.experimental.pallas import tpu_sc as plsc

D = 128
DE = 144
NC, NS, LN = 2, 16, 16
CHUNK = 64
NBLK = 2000


def _ceil_div(a, b):
    return (a + b - 1) // b


def _mm_body(x_ref, w_ref, as_ref, ad_ref, he_ref, sd_ref):
    i = pl.program_id(0)
    h = jnp.dot(x_ref[...], w_ref[...], preferred_element_type=jnp.float32)
    he_ref[:, :D] = h
    lane = lax.broadcasted_iota(jnp.int32, (he_ref.shape[0], DE - D), 1)
    he_ref[:, D:] = jnp.where(lane == 0, 1.0, 0.0)
    s = jnp.dot(h, as_ref[0, :], preferred_element_type=jnp.float32)
    d = jnp.dot(h, ad_ref[0, :], preferred_element_type=jnp.float32)
    sd_ref[0] = jnp.stack([s, d])


def _mm_norm_body(x_ref, mustd_ref, w_ref, as_ref, ad_ref, he_ref, sd_ref):
    i = pl.program_id(0)
    xn = (x_ref[...] - mustd_ref[0, :]) * mustd_ref[1, :]
    xn = jnp.maximum(xn, 0.0)
    h = jnp.dot(xn, w_ref[...], preferred_element_type=jnp.float32)
    he_ref[:, :D] = h
    lane = lax.broadcasted_iota(jnp.int32, (he_ref.shape[0], DE - D), 1)
    he_ref[:, D:] = jnp.where(lane == 0, 1.0, 0.0)
    s = jnp.dot(h, as_ref[0, :], preferred_element_type=jnp.float32)
    d = jnp.dot(h, ad_ref[0, :], preferred_element_type=jnp.float32)
    sd_ref[0] = jnp.stack([s, d])


def _project(x, W, a_s, a_d, mustd=None):
    n = x.shape[0]
    grid = (n // NBLK,)
    in_specs = [pl.BlockSpec((NBLK, D), lambda i: (i, 0))]
    args = [x]
    body = _mm_body
    if mustd is not None:
        in_specs.append(pl.BlockSpec((2, D), lambda i: (0, 0)))
        args.append(mustd)
        body = _mm_norm_body
    in_specs += [
        pl.BlockSpec((D, D), lambda i: (0, 0)),
        pl.BlockSpec((1, D), lambda i: (0, 0)),
        pl.BlockSpec((1, D), lambda i: (0, 0)),
    ]
    args += [W, a_s.reshape(1, D), a_d.reshape(1, D)]
    return pl.pallas_call(
        body,
        grid=grid,
        in_specs=in_specs,
        out_specs=[
            pl.BlockSpec((NBLK, DE), lambda i: (i, 0)),
            pl.BlockSpec((1, 2, NBLK), lambda i: (i, 0, 0)),
        ],
        out_shape=[
            jax.ShapeDtypeStruct((n, DE), jnp.float32),
            jax.ShapeDtypeStruct((n // NBLK, 2, NBLK), jnp.float32),
        ],
    )(*args)


def _et_body(ea_ref, ec_ref, et_ref, sum_ref, acc):
    i = pl.program_id(0)
    v = jnp.dot(ea_ref[...], ec_ref[0, :], preferred_element_type=jnp.float32)
    et_ref[0, :] = v

    @pl.when(i == 0)
    def _():
        acc[...] = jnp.zeros_like(acc)

    acc[0, :] += jnp.sum(v.reshape(-1, D), axis=0)
    sum_ref[...] = acc[...]


def _edge_terms(edge_att, ec):
    e = edge_att.shape[0]
    blk = 12800
    return pl.pallas_call(
        _et_body,
        grid=(e // blk,),
        in_specs=[
            pl.BlockSpec((blk, ec.shape[0] if False else 16), lambda i: (i, 0)),
            pl.BlockSpec((1, 16), lambda i: (0, 0)),
        ],
        out_specs=[
            pl.BlockSpec((1, blk), lambda i: (0, i)),
            pl.BlockSpec((1, D), lambda i: (0, 0)),
        ],
        out_shape=[
            jax.ShapeDtypeStruct((1, e), jnp.float32),
            jax.ShapeDtypeStruct((1, D), jnp.float32),
        ],
        scratch_shapes=[pltpu.VMEM((1, D), jnp.float32)],
    )(edge_att, ec.reshape(1, 16))


def _merge_body(a0_ref, a1_ref, b_ref, out_ref, stats_ref, acc):
    i = pl.program_id(0)
    a0 = a0_ref[...]
    a1 = a1_ref[...]
    num = a0[:, :D] + a1[:, :D]
    den = a0[:, D] + a1[:, D]
    out = num / (den[:, None] + 1e-16) + b_ref[0, :]
    out_ref[...] = out

    @pl.when(i == 0)
    def _():
        acc[...] = jnp.zeros_like(acc)

    acc[0, :] += jnp.sum(out, axis=0)
    acc[1, :] += jnp.sum(out * out, axis=0)
    stats_ref[...] = acc[...]


def _merge(acc_a, acc_b, b, n):
    return pl.pallas_call(
        _merge_body,
        grid=(n // NBLK,),
        in_specs=[
            pl.BlockSpec((NBLK, DE), lambda i: (i, 0)),
            pl.BlockSpec((NBLK, DE), lambda i: (i, 0)),
            pl.BlockSpec((1, D), lambda i: (0, 0)),
        ],
        out_specs=[
            pl.BlockSpec((NBLK, D), lambda i: (i, 0)),
            pl.BlockSpec((2, D), lambda i: (0, 0)),
        ],
        out_shape=[
            jax.ShapeDtypeStruct((n, D), jnp.float32),
            jax.ShapeDtypeStruct((2, D), jnp.float32),
        ],
        scratch_shapes=[pltpu.VMEM((2, D), jnp.float32)],
    )(acc_a, acc_b, b.reshape(1, D))


def _norm_body(x_ref, mustd_ref, o_ref):
    o_ref[...] = (x_ref[...] - mustd_ref[0, :]) * mustd_ref[1, :]


def _normalize(x, mustd):
    n = x.shape[0]
    return pl.pallas_call(
        _norm_body,
        grid=(n // NBLK,),
        in_specs=[
            pl.BlockSpec((NBLK, D), lambda i: (i, 0)),
            pl.BlockSpec((2, D), lambda i: (0, 0)),
        ],
        out_specs=pl.BlockSpec((NBLK, D), lambda i: (i, 0)),
        out_shape=jax.ShapeDtypeStruct((n, D), jnp.float32),
    )(x, mustd)


def _sc_edge_pass(h_ext, sd, src2, dst2, et2, n, nch):
    ngrp = nch // 4
    rsub = (n // NS) // 8 * 8
    rlast = n - (NS - 1) * rsub
    nz = rsub // 48

    mesh = plsc.VectorSubcoreMesh(
        core_axis_name="c", subcore_axis_name="s", num_cores=1)
    cp = pltpu.CompilerParams()
    for fld, val in (("needs_layout_passes", False),
                     ("use_tc_tiling_on_sc", False)):
        if fld in pltpu.CompilerParams.__dataclass_fields__:
            cp = dataclasses.replace(cp, **{fld: val})

    @functools.partial(
        pl.kernel,
        out_type=jax.ShapeDtypeStruct((n, DE), jnp.float32),
        mesh=mesh,
        compiler_params=cp,
        scratch_types=[
            pltpu.VMEM((2, n), jnp.float32),
            pltpu.VMEM((2, CHUNK), jnp.int32),
            pltpu.VMEM((2, CHUNK), jnp.int32),
            pltpu.VMEM((2, CHUNK), jnp.float32),
            pltpu.VMEM((2, CHUNK), jnp.int32),
            pltpu.VMEM((2, CHUNK), jnp.int32),
            pltpu.VMEM((2, CHUNK), jnp.float32),
            pltpu.VMEM((CHUNK, DE), jnp.float32),
            pltpu.VMEM((CHUNK, DE), jnp.float32),
            pltpu.SemaphoreType.DMA,
            pltpu.SemaphoreType.DMA,
            pltpu.SemaphoreType.DMA,
            pltpu.SemaphoreType.DMA,
            pltpu.SemaphoreType.DMA,
            pltpu.SemaphoreType.DMA,
            pltpu.VMEM_SHARED((n, DE), jnp.float32),
        ],
    )
    def k(h_hbm, sd_hbm, src_hbm, dst_hbm, et_hbm, out_hbm,
          sd_v, srcA, dstA, etA, srcB, dstB, etB, rowsA, rowsB,
          gA, gB, sA, sB, wA, wB, acc_sh):
        sid = lax.axis_index("s")

        zero16 = jnp.zeros((LN,), jnp.float32)

        @pl.loop(0, 48)
        def _(r):
            for cc in range(DE // LN):
                rowsA[r, pl.ds(cc * LN, LN)] = zero16

        pltpu.sync_copy(sd_hbm, sd_v)

        base = sid * rsub

        @pl.loop(0, nz)
        def _(j):
            pltpu.sync_copy(rowsA.at[pl.ds(0, 48)],
                            acc_sh.at[pl.ds(base + j * 48, 48)])

        @pl.when(sid == NS - 1)
        def _():
            pltpu.sync_copy(rowsA.at[pl.ds(0, 16)],
                            acc_sh.at[pl.ds(base + nz * 48, 16)])

        plsc.subcore_barrier()

        zeros_i = jnp.zeros((LN,), jnp.int32)
        ones_i = jnp.ones((LN,), jnp.int32)

        def compute(rows_v, srcw, dstw, etw, r):
            @pl.loop(0, CHUNK // LN)
            def _(g):
                isv = srcw[r, pl.ds(g * LN, LN)]
                idv = dstw[r, pl.ds(g * LN, LN)]
                sg = plsc.load_gather(sd_v, [zeros_i, isv])
                dg = plsc.load_gather(sd_v, [ones_i, idv])
                a = sg + dg + etw[r, pl.ds(g * LN, LN)]
                a = jnp.maximum(a, a * 0.2)
                exv = jnp.exp(a)
                for j in range(LN):
                    w = exv[j]
                    e = g * LN + j
                    for rr in range(DE // LN):
                        slc = (e, pl.ds(rr * LN, LN))
                        rows_v[slc] = rows_v[slc] * w

        def win_fetch(p, dst3, sem):
            sw, dw, ew = dst3
            pltpu.async_copy(src_hbm.at[sid, pl.ds(p * 2, 2)], sw, sem)
            pltpu.async_copy(dst_hbm.at[sid, pl.ds(p * 2, 2)], dw, sem)
            pltpu.async_copy(et_hbm.at[sid, pl.ds(p * 2, 2)], ew, sem)

        def win_wait(p, dst3, sem):
            sw, dw, ew = dst3
            pltpu.make_async_copy(src_hbm.at[sid, pl.ds(p * 2, 2)], sw, sem).wait()
            pltpu.make_async_copy(dst_hbm.at[sid, pl.ds(p * 2, 2)], dw, sem).wait()
            pltpu.make_async_copy(et_hbm.at[sid, pl.ds(p * 2, 2)], ew, sem).wait()

        def gather(srcw, r, rows_v, sem):
            pltpu.async_copy(h_hbm.at[srcw.at[r]], rows_v, sem)

        def gather_wait(srcw, r, rows_v, sem):
            pltpu.make_async_copy(h_hbm.at[srcw.at[r]], rows_v, sem).wait()

        def scatter(rows_v, dstw, r, sem):
            pltpu.async_copy(rows_v, acc_sh.at[dstw.at[r]], sem, add=True)

        def scatter_wait(rows_v, dstw, r, sem):
            pltpu.make_async_copy(rows_v, acc_sh.at[dstw.at[r]], sem).wait()

        winA = (srcA, dstA, etA)
        winB = (srcB, dstB, etB)

        win_fetch(0, winA, wA)
        win_wait(0, winA, wA)
        win_fetch(1, winB, wB)
        gather(srcA, 0, rowsA, gA)
        gather(srcA, 1, rowsB, gB)

        @pl.loop(0, ngrp - 1)
        def _(grp):
            gather_wait(srcA, 0, rowsA, gA)
            compute(rowsA, srcA, dstA, etA, 0)
            scatter(rowsA, dstA, 0, sA)
            gather_wait(srcA, 1, rowsB, gB)
            compute(rowsB, srcA, dstA, etA, 1)
            scatter(rowsB, dstA, 1, sB)
            win_wait(2 * grp + 1, winB, wB)
            scatter_wait(rowsA, dstA, 0, sA)
            gather(srcB, 0, rowsA, gA)
            scatter_wait(rowsB, dstA, 1, sB)
            gather(srcB, 1, rowsB, gB)
            win_fetch(2 * grp + 2, winA, wA)
            gather_wait(srcB, 0, rowsA, gA)
            compute(rowsA, srcB, dstB, etB, 0)
            scatter(rowsA, dstB, 0, sA)
            gather_wait(srcB, 1, rowsB, gB)
            compute(rowsB, srcB, dstB, etB, 1)
            scatter(rowsB, dstB, 1, sB)
            win_wait(2 * grp + 2, winA, wA)
            scatter_wait(rowsA, dstB, 0, sA)
            gather(srcA, 0, rowsA, gA)
            scatter_wait(rowsB, dstB, 1, sB)
            gather(srcA, 1, rowsB, gB)
            win_fetch(2 * grp + 3, winB, wB)

        gather_wait(srcA, 0, rowsA, gA)
        compute(rowsA, srcA, dstA, etA, 0)
        scatter(rowsA, dstA, 0, sA)
        gather_wait(srcA, 1, rowsB, gB)
        compute(rowsB, srcA, dstA, etA, 1)
        scatter(rowsB, dstA, 1, sB)
        win_wait(2 * ngrp - 1, winB, wB)
        scatter_wait(rowsA, dstA, 0, sA)
        gather(srcB, 0, rowsA, gA)
        scatter_wait(rowsB, dstA, 1, sB)
        gather(srcB, 1, rowsB, gB)
        gather_wait(srcB, 0, rowsA, gA)
        compute(rowsA, srcB, dstB, etB, 0)
        scatter(rowsA, dstB, 0, sA)
        gather_wait(srcB, 1, rowsB, gB)
        compute(rowsB, srcB, dstB, etB, 1)
        scatter(rowsB, dstB, 1, sB)
        scatter_wait(rowsA, dstB, 0, sA)
        scatter_wait(rowsB, dstB, 1, sB)

        plsc.subcore_barrier()

        pltpu.sync_copy(acc_sh.at[pl.ds(base, rsub)],
                        out_hbm.at[pl.ds(base, rsub)])

        @pl.when(sid == NS - 1)
        def _():
            pltpu.sync_copy(acc_sh.at[pl.ds(base + rsub, rlast - rsub)],
                            out_hbm.at[pl.ds(base + rsub, rlast - rsub)])

    return k(h_ext, sd, src2, dst2, et2)


def kernel(x, edge_index, edge_att, W1, We1, as1, ad1, ae1, b1,
           W2, We2, as2, ad2, ae2, b2):
    n = x.shape[0]
    e = edge_index.shape[1]
    ep_total = e + n
    nw = NS
    nch = _ceil_div(_ceil_div(ep_total, nw * CHUNK), 8) * 8
    ep = nw * CHUNK * nch
    nch2 = nch // 2
    pad = ep - ep_total

    loops = jnp.arange(n, dtype=jnp.int32)
    src = jnp.concatenate(
        [edge_index[0].astype(jnp.int32), loops, jnp.zeros((pad,), jnp.int32)])
    dst = jnp.concatenate(
        [edge_index[1].astype(jnp.int32), loops, jnp.zeros((pad,), jnp.int32)])
    src2 = src.reshape(2, nw, nch2, CHUNK)
    dst2 = dst.reshape(2, nw, nch2, CHUNK)

    ec1 = We1 @ ae1
    ec2 = We2 @ ae2
    et1_main, et1_sum = _edge_terms(edge_att, ec1)
    et2_main, et2_sum = _edge_terms(edge_att, ec2)

    def pack_et(et_main, et_sum):
        self_term = jnp.broadcast_to(jnp.sum(et_sum) / e, (n,))
        et = jnp.concatenate(
            [et_main[0], self_term, jnp.full((pad,), -1e30, jnp.float32)])
        return et.reshape(2, nw, nch2, CHUNK)

    et1 = pack_et(et1_main, et1_sum)
    et2 = pack_et(et2_main, et2_sum)

    def bn_mustd(stats):
        mu = stats[0] / n
        var = stats[1] / n - mu * mu
        return jnp.stack([mu, 1.0 / jnp.sqrt(var + 1e-5)])

    h1e, sd1 = _project(x, W1, as1, ad1)
    sd1 = sd1.transpose(1, 0, 2).reshape(2, n)
    acc1a = _sc_edge_pass(h1e, sd1, src2[0], dst2[0], et1[0], n, nch2)
    acc1b = _sc_edge_pass(h1e, sd1, src2[1], dst2[1], et1[1], n, nch2)
    out1, stats1 = _merge(acc1a, acc1b, b1, n)

    h2e, sd2 = _project(out1, W2, as2, ad2, mustd=bn_mustd(stats1))
    sd2 = sd2.transpose(1, 0, 2).reshape(2, n)
    acc2a = _sc_edge_pass(h2e, sd2, src2[0], dst2[0], et2[0], n, nch2)
    acc2b = _sc_edge_pass(h2e, sd2, src2[1], dst2[1], et2[1], n, nch2)
    out2, stats2 = _merge(acc2a, acc2b, b2, n)

    return _normalize(out2, bn_mustd(stats2))


# speedup vs baseline: 1.2534x; 1.2174x over previous
import dataclasses
import functools

import jax
import jax.numpy as jnp
from jax import lax
from jax.experimental import pallas as pl
from jax.experimental.pallas import tpu as pltpu
from jax.experimental.pallas import tpu_sc as plsc

D = 128
DE = 144
NC, NS, LN = 2, 16, 16
CHUNK = 64
NBLK = 2000


def _ceil_div(a, b):
    return (a + b - 1) // b


def _mm_body(x_ref, w_ref, as_ref, ad_ref, he_ref, sd_ref):
    i = pl.program_id(0)
    h = jnp.dot(x_ref[...], w_ref[...], preferred_element_type=jnp.float32)
    he_ref[:, :D] = h
    lane = lax.broadcasted_iota(jnp.int32, (he_ref.shape[0], DE - D), 1)
    he_ref[:, D:] = jnp.where(lane == 0, 1.0, 0.0)
    s = jnp.dot(h, as_ref[0, :], preferred_element_type=jnp.float32)
    d = jnp.dot(h, ad_ref[0, :], preferred_element_type=jnp.float32)
    sd_ref[0] = jnp.stack([s, d])


def _mm_norm_body(x_ref, mustd_ref, w_ref, as_ref, ad_ref, he_ref, sd_ref):
    i = pl.program_id(0)
    xn = (x_ref[...] - mustd_ref[0, :]) * mustd_ref[1, :]
    xn = jnp.maximum(xn, 0.0)
    h = jnp.dot(xn, w_ref[...], preferred_element_type=jnp.float32)
    he_ref[:, :D] = h
    lane = lax.broadcasted_iota(jnp.int32, (he_ref.shape[0], DE - D), 1)
    he_ref[:, D:] = jnp.where(lane == 0, 1.0, 0.0)
    s = jnp.dot(h, as_ref[0, :], preferred_element_type=jnp.float32)
    d = jnp.dot(h, ad_ref[0, :], preferred_element_type=jnp.float32)
    sd_ref[0] = jnp.stack([s, d])


def _project(x, W, a_s, a_d, mustd=None):
    n = x.shape[0]
    grid = (n // NBLK,)
    in_specs = [pl.BlockSpec((NBLK, D), lambda i: (i, 0))]
    args = [x]
    body = _mm_body
    if mustd is not None:
        in_specs.append(pl.BlockSpec((2, D), lambda i: (0, 0)))
        args.append(mustd)
        body = _mm_norm_body
    in_specs += [
        pl.BlockSpec((D, D), lambda i: (0, 0)),
        pl.BlockSpec((1, D), lambda i: (0, 0)),
        pl.BlockSpec((1, D), lambda i: (0, 0)),
    ]
    args += [W, a_s.reshape(1, D), a_d.reshape(1, D)]
    return pl.pallas_call(
        body,
        grid=grid,
        in_specs=in_specs,
        out_specs=[
            pl.BlockSpec((NBLK, DE), lambda i: (i, 0)),
            pl.BlockSpec((1, 2, NBLK), lambda i: (i, 0, 0)),
        ],
        out_shape=[
            jax.ShapeDtypeStruct((n, DE), jnp.float32),
            jax.ShapeDtypeStruct((n // NBLK, 2, NBLK), jnp.float32),
        ],
    )(*args)


def _et_body(ea_ref, ec_ref, et1_ref, et2_ref, sum_ref, acc):
    i = pl.program_id(0)
    ea = ea_ref[...]
    v1 = jnp.dot(ea, ec_ref[0, :], preferred_element_type=jnp.float32)
    v2 = jnp.dot(ea, ec_ref[1, :], preferred_element_type=jnp.float32)
    et1_ref[0, :] = v1
    et2_ref[0, :] = v2

    @pl.when(i == 0)
    def _():
        acc[...] = jnp.zeros_like(acc)

    acc[0, :] += jnp.sum(v1.reshape(-1, D), axis=0)
    acc[1, :] += jnp.sum(v2.reshape(-1, D), axis=0)
    sum_ref[...] = acc[...]


def _edge_terms(edge_att, ec1, ec2):
    e = edge_att.shape[0]
    blk = 12800
    return pl.pallas_call(
        _et_body,
        grid=(e // blk,),
        in_specs=[
            pl.BlockSpec((blk, 16), lambda i: (i, 0)),
            pl.BlockSpec((2, 16), lambda i: (0, 0)),
        ],
        out_specs=[
            pl.BlockSpec((1, blk), lambda i: (0, i)),
            pl.BlockSpec((1, blk), lambda i: (0, i)),
            pl.BlockSpec((2, D), lambda i: (0, 0)),
        ],
        out_shape=[
            jax.ShapeDtypeStruct((1, e), jnp.float32),
            jax.ShapeDtypeStruct((1, e), jnp.float32),
            jax.ShapeDtypeStruct((2, D), jnp.float32),
        ],
        scratch_shapes=[pltpu.VMEM((2, D), jnp.float32)],
    )(edge_att, jnp.stack([ec1, ec2]))


def _merge_body(a0_ref, b_ref, out_ref, stats_ref, acc):
    i = pl.program_id(0)
    a0 = a0_ref[...]
    num = a0[:, :D]
    den = a0[:, D]
    out = num / (den[:, None] + 1e-16) + b_ref[0, :]
    out_ref[...] = out

    @pl.when(i == 0)
    def _():
        acc[...] = jnp.zeros_like(acc)

    acc[0, :] += jnp.sum(out, axis=0)
    acc[1, :] += jnp.sum(out * out, axis=0)
    stats_ref[...] = acc[...]


def _merge(acc, b, n):
    return pl.pallas_call(
        _merge_body,
        grid=(n // NBLK,),
        in_specs=[
            pl.BlockSpec((NBLK, DE), lambda i: (i, 0)),
            pl.BlockSpec((1, D), lambda i: (0, 0)),
        ],
        out_specs=[
            pl.BlockSpec((NBLK, D), lambda i: (i, 0)),
            pl.BlockSpec((2, D), lambda i: (0, 0)),
        ],
        out_shape=[
            jax.ShapeDtypeStruct((n, D), jnp.float32),
            jax.ShapeDtypeStruct((2, D), jnp.float32),
        ],
        scratch_shapes=[pltpu.VMEM((2, D), jnp.float32)],
    )(acc, b.reshape(1, D))


def _norm_body(x_ref, mustd_ref, o_ref):
    o_ref[...] = (x_ref[...] - mustd_ref[0, :]) * mustd_ref[1, :]


def _normalize(x, mustd):
    n = x.shape[0]
    return pl.pallas_call(
        _norm_body,
        grid=(n // NBLK,),
        in_specs=[
            pl.BlockSpec((NBLK, D), lambda i: (i, 0)),
            pl.BlockSpec((2, D), lambda i: (0, 0)),
        ],
        out_specs=pl.BlockSpec((NBLK, D), lambda i: (i, 0)),
        out_shape=jax.ShapeDtypeStruct((n, D), jnp.float32),
    )(x, mustd)


def _sc_edge_pass(h_ext, sd, src2, dst2, et2, n, nch):
    ngrp = nch // 4
    rsub = (n // NS) // 8 * 8
    rlast = n - (NS - 1) * rsub
    nz = rsub // 48

    mesh = plsc.VectorSubcoreMesh(
        core_axis_name="c", subcore_axis_name="s", num_cores=1)
    cp = pltpu.CompilerParams()
    for fld, val in (("needs_layout_passes", False),
                     ("use_tc_tiling_on_sc", False)):
        if fld in pltpu.CompilerParams.__dataclass_fields__:
            cp = dataclasses.replace(cp, **{fld: val})

    @functools.partial(
        pl.kernel,
        out_type=jax.ShapeDtypeStruct((n, DE), jnp.float32),
        mesh=mesh,
        compiler_params=cp,
        scratch_types=[
            pltpu.VMEM((2, n), jnp.float32),
            pltpu.VMEM((2, CHUNK), jnp.int32),
            pltpu.VMEM((2, CHUNK), jnp.int32),
            pltpu.VMEM((2, CHUNK), jnp.float32),
            pltpu.VMEM((2, CHUNK), jnp.int32),
            pltpu.VMEM((2, CHUNK), jnp.int32),
            pltpu.VMEM((2, CHUNK), jnp.float32),
            pltpu.VMEM((CHUNK, DE), jnp.float32),
            pltpu.VMEM((CHUNK, DE), jnp.float32),
            pltpu.SemaphoreType.DMA,
            pltpu.SemaphoreType.DMA,
            pltpu.SemaphoreType.DMA,
            pltpu.SemaphoreType.DMA,
            pltpu.SemaphoreType.DMA,
            pltpu.SemaphoreType.DMA,
            pltpu.VMEM_SHARED((n, DE), jnp.float32),
        ],
    )
    def k(h_hbm, sd_hbm, src_hbm, dst_hbm, et_hbm, out_hbm,
          sd_v, srcA, dstA, etA, srcB, dstB, etB, rowsA, rowsB,
          gA, gB, sA, sB, wA, wB, acc_sh):
        sid = lax.axis_index("s")

        zero16 = jnp.zeros((LN,), jnp.float32)

        @pl.loop(0, 48)
        def _(r):
            for cc in range(DE // LN):
                rowsA[r, pl.ds(cc * LN, LN)] = zero16

        pltpu.sync_copy(sd_hbm, sd_v)

        base = sid * rsub

        @pl.loop(0, nz)
        def _(j):
            pltpu.sync_copy(rowsA.at[pl.ds(0, 48)],
                            acc_sh.at[pl.ds(base + j * 48, 48)])

        @pl.when(sid == NS - 1)
        def _():
            pltpu.sync_copy(rowsA.at[pl.ds(0, 16)],
                            acc_sh.at[pl.ds(base + nz * 48, 16)])

        plsc.subcore_barrier()

        zeros_i = jnp.zeros((LN,), jnp.int32)
        ones_i = jnp.ones((LN,), jnp.int32)

        def compute(rows_v, srcw, dstw, etw, r):
            @pl.loop(0, CHUNK // LN)
            def _(g):
                isv = srcw[r, pl.ds(g * LN, LN)]
                idv = dstw[r, pl.ds(g * LN, LN)]
                sg = plsc.load_gather(sd_v, [zeros_i, isv])
                dg = plsc.load_gather(sd_v, [ones_i, idv])
                a = sg + dg + etw[r, pl.ds(g * LN, LN)]
                a = jnp.maximum(a, a * 0.2)
                exv = jnp.exp(a)
                for j in range(LN):
                    w = exv[j]
                    e = g * LN + j
                    for rr in range(DE // LN):
                        slc = (e, pl.ds(rr * LN, LN))
                        rows_v[slc] = rows_v[slc] * w

        def win_fetch(p, dst3, sem):
            sw, dw, ew = dst3
            pltpu.async_copy(src_hbm.at[sid, pl.ds(p * 2, 2)], sw, sem)
            pltpu.async_copy(dst_hbm.at[sid, pl.ds(p * 2, 2)], dw, sem)
            pltpu.async_copy(et_hbm.at[sid, pl.ds(p * 2, 2)], ew, sem)

        def win_wait(p, dst3, sem):
            sw, dw, ew = dst3
            pltpu.make_async_copy(src_hbm.at[sid, pl.ds(p * 2, 2)], sw, sem).wait()
            pltpu.make_async_copy(dst_hbm.at[sid, pl.ds(p * 2, 2)], dw, sem).wait()
            pltpu.make_async_copy(et_hbm.at[sid, pl.ds(p * 2, 2)], ew, sem).wait()

        def gather(srcw, r, rows_v, sem):
            pltpu.async_copy(h_hbm.at[srcw.at[r]], rows_v, sem)

        def gather_wait(srcw, r, rows_v, sem):
            pltpu.make_async_copy(h_hbm.at[srcw.at[r]], rows_v, sem).wait()

        def scatter(rows_v, dstw, r, sem):
            pltpu.async_copy(rows_v, acc_sh.at[dstw.at[r]], sem, add=True)

        def scatter_wait(rows_v, dstw, r, sem):
            pltpu.make_async_copy(rows_v, acc_sh.at[dstw.at[r]], sem).wait()

        winA = (srcA, dstA, etA)
        winB = (srcB, dstB, etB)

        win_fetch(0, winA, wA)
        win_wait(0, winA, wA)
        win_fetch(1, winB, wB)
        gather(srcA, 0, rowsA, gA)
        gather(srcA, 1, rowsB, gB)

        @pl.loop(0, ngrp - 1)
        def _(grp):
            gather_wait(srcA, 0, rowsA, gA)
            compute(rowsA, srcA, dstA, etA, 0)
            scatter(rowsA, dstA, 0, sA)
            gather_wait(srcA, 1, rowsB, gB)
            compute(rowsB, srcA, dstA, etA, 1)
            scatter(rowsB, dstA, 1, sB)
            win_wait(2 * grp + 1, winB, wB)
            scatter_wait(rowsA, dstA, 0, sA)
            gather(srcB, 0, rowsA, gA)
            scatter_wait(rowsB, dstA, 1, sB)
            gather(srcB, 1, rowsB, gB)
            win_fetch(2 * grp + 2, winA, wA)
            gather_wait(srcB, 0, rowsA, gA)
            compute(rowsA, srcB, dstB, etB, 0)
            scatter(rowsA, dstB, 0, sA)
            gather_wait(srcB, 1, rowsB, gB)
            compute(rowsB, srcB, dstB, etB, 1)
            scatter(rowsB, dstB, 1, sB)
            win_wait(2 * grp + 2, winA, wA)
            scatter_wait(rowsA, dstB, 0, sA)
            gather(srcA, 0, rowsA, gA)
            scatter_wait(rowsB, dstB, 1, sB)
            gather(srcA, 1, rowsB, gB)
            win_fetch(2 * grp + 3, winB, wB)

        gather_wait(srcA, 0, rowsA, gA)
        compute(rowsA, srcA, dstA, etA, 0)
        scatter(rowsA, dstA, 0, sA)
        gather_wait(srcA, 1, rowsB, gB)
        compute(rowsB, srcA, dstA, etA, 1)
        scatter(rowsB, dstA, 1, sB)
        win_wait(2 * ngrp - 1, winB, wB)
        scatter_wait(rowsA, dstA, 0, sA)
        gather(srcB, 0, rowsA, gA)
        scatter_wait(rowsB, dstA, 1, sB)
        gather(srcB, 1, rowsB, gB)
        gather_wait(srcB, 0, rowsA, gA)
        compute(rowsA, srcB, dstB, etB, 0)
        scatter(rowsA, dstB, 0, sA)
        gather_wait(srcB, 1, rowsB, gB)
        compute(rowsB, srcB, dstB, etB, 1)
        scatter(rowsB, dstB, 1, sB)
        scatter_wait(rowsA, dstB, 0, sA)
        scatter_wait(rowsB, dstB, 1, sB)

        plsc.subcore_barrier()

        pltpu.sync_copy(acc_sh.at[pl.ds(base, rsub)],
                        out_hbm.at[pl.ds(base, rsub)])

        @pl.when(sid == NS - 1)
        def _():
            pltpu.sync_copy(acc_sh.at[pl.ds(base + rsub, rlast - rsub)],
                            out_hbm.at[pl.ds(base + rsub, rlast - rsub)])

    return k(h_ext, sd, src2, dst2, et2)


def kernel(x, edge_index, edge_att, W1, We1, as1, ad1, ae1, b1,
           W2, We2, as2, ad2, ae2, b2):
    n = x.shape[0]
    e = edge_index.shape[1]
    ep_total = e + n
    nw = NS
    nch = _ceil_div(_ceil_div(ep_total, nw * CHUNK), 4) * 4
    ep = nw * CHUNK * nch
    pad = ep - ep_total

    loops = jnp.arange(n, dtype=jnp.int32)
    src = jnp.concatenate(
        [edge_index[0].astype(jnp.int32), loops, jnp.zeros((pad,), jnp.int32)])
    dst = jnp.concatenate(
        [edge_index[1].astype(jnp.int32), loops, jnp.zeros((pad,), jnp.int32)])
    src2 = src.reshape(nw, nch, CHUNK)
    dst2 = dst.reshape(nw, nch, CHUNK)

    ec1 = We1 @ ae1
    ec2 = We2 @ ae2
    et1_main, et2_main, et_sums = _edge_terms(edge_att, ec1, ec2)

    def pack_et(et_main, et_sum):
        self_term = jnp.broadcast_to(jnp.sum(et_sum) / e, (n,))
        et = jnp.concatenate(
            [et_main[0], self_term, jnp.full((pad,), -1e30, jnp.float32)])
        return et.reshape(nw, nch, CHUNK)

    et1 = pack_et(et1_main, et_sums[0])
    et2 = pack_et(et2_main, et_sums[1])

    def bn_mustd(stats):
        mu = stats[0] / n
        var = stats[1] / n - mu * mu
        return jnp.stack([mu, 1.0 / jnp.sqrt(var + 1e-5)])

    h1e, sd1 = _project(x, W1, as1, ad1)
    sd1 = sd1.transpose(1, 0, 2).reshape(2, n)
    acc1 = _sc_edge_pass(h1e, sd1, src2, dst2, et1, n, nch)
    out1, stats1 = _merge(acc1, b1, n)

    h2e, sd2 = _project(out1, W2, as2, ad2, mustd=bn_mustd(stats1))
    sd2 = sd2.transpose(1, 0, 2).reshape(2, n)
    acc2 = _sc_edge_pass(h2e, sd2, src2, dst2, et2, n, nch)
    out2, stats2 = _merge(acc2, b2, n)

    return _normalize(out2, bn_mustd(stats2))


# speedup vs baseline: 1.4182x; 1.1315x over previous
import dataclasses
import functools

import jax
import jax.numpy as jnp
from jax import lax
from jax.experimental import pallas as pl
from jax.experimental.pallas import tpu as pltpu
from jax.experimental.pallas import tpu_sc as plsc

D = 128
DE = 144
NC, NS, LN = 2, 16, 16
CHUNK = 64
NBLK = 2000


def _ceil_div(a, b):
    return (a + b - 1) // b


def _mm_body(x_ref, w_ref, as_ref, ad_ref, he_ref, sd_ref):
    i = pl.program_id(0)
    h = jnp.dot(x_ref[...], w_ref[...], preferred_element_type=jnp.float32)
    he_ref[:, :D] = h
    lane = lax.broadcasted_iota(jnp.int32, (he_ref.shape[0], DE - D), 1)
    he_ref[:, D:] = jnp.where(lane == 0, 1.0, 0.0)
    s = jnp.dot(h, as_ref[0, :], preferred_element_type=jnp.float32)
    d = jnp.dot(h, ad_ref[0, :], preferred_element_type=jnp.float32)
    sd_ref[0] = jnp.stack([s, d])


def _mm_norm_body(x_ref, mustd_ref, w_ref, as_ref, ad_ref, he_ref, sd_ref):
    i = pl.program_id(0)
    xn = (x_ref[...] - mustd_ref[0, :]) * mustd_ref[1, :]
    xn = jnp.maximum(xn, 0.0)
    h = jnp.dot(xn, w_ref[...], preferred_element_type=jnp.float32)
    he_ref[:, :D] = h
    lane = lax.broadcasted_iota(jnp.int32, (he_ref.shape[0], DE - D), 1)
    he_ref[:, D:] = jnp.where(lane == 0, 1.0, 0.0)
    s = jnp.dot(h, as_ref[0, :], preferred_element_type=jnp.float32)
    d = jnp.dot(h, ad_ref[0, :], preferred_element_type=jnp.float32)
    sd_ref[0] = jnp.stack([s, d])


def _project(x, W, a_s, a_d, mustd=None):
    n = x.shape[0]
    grid = (n // NBLK,)
    in_specs = [pl.BlockSpec((NBLK, D), lambda i: (i, 0))]
    args = [x]
    body = _mm_body
    if mustd is not None:
        in_specs.append(pl.BlockSpec((2, D), lambda i: (0, 0)))
        args.append(mustd)
        body = _mm_norm_body
    in_specs += [
        pl.BlockSpec((D, D), lambda i: (0, 0)),
        pl.BlockSpec((1, D), lambda i: (0, 0)),
        pl.BlockSpec((1, D), lambda i: (0, 0)),
    ]
    args += [W, a_s.reshape(1, D), a_d.reshape(1, D)]
    return pl.pallas_call(
        body,
        grid=grid,
        in_specs=in_specs,
        out_specs=[
            pl.BlockSpec((NBLK, DE), lambda i: (i, 0)),
            pl.BlockSpec((1, 2, NBLK), lambda i: (i, 0, 0)),
        ],
        out_shape=[
            jax.ShapeDtypeStruct((n, DE), jnp.float32),
            jax.ShapeDtypeStruct((n // NBLK, 2, NBLK), jnp.float32),
        ],
    )(*args)


def _et_body(ea_ref, ec_ref, et_ref, sum_ref, acc):
    i = pl.program_id(0)
    v = jnp.dot(ea_ref[...], ec_ref[0, :], preferred_element_type=jnp.float32)
    et_ref[0, :] = v

    @pl.when(i == 0)
    def _():
        acc[...] = jnp.zeros_like(acc)

    acc[0, :] += jnp.sum(v.reshape(-1, D), axis=0)
    sum_ref[...] = acc[...]


def _edge_terms(edge_att, ec):
    e = edge_att.shape[0]
    blk = 12800
    return pl.pallas_call(
        _et_body,
        grid=(e // blk,),
        in_specs=[
            pl.BlockSpec((blk, ec.shape[0] if False else 16), lambda i: (i, 0)),
            pl.BlockSpec((1, 16), lambda i: (0, 0)),
        ],
        out_specs=[
            pl.BlockSpec((1, blk), lambda i: (0, i)),
            pl.BlockSpec((1, D), lambda i: (0, 0)),
        ],
        out_shape=[
            jax.ShapeDtypeStruct((1, e), jnp.float32),
            jax.ShapeDtypeStruct((1, D), jnp.float32),
        ],
        scratch_shapes=[pltpu.VMEM((1, D), jnp.float32)],
    )(edge_att, ec.reshape(1, 16))


def _merge_body(a0_ref, b_ref, out_ref, stats_ref, acc):
    i = pl.program_id(0)
    a0 = a0_ref[...]
    num = a0[:, :D]
    den = a0[:, D]
    out = num / (den[:, None] + 1e-16) + b_ref[0, :]
    out_ref[...] = out

    @pl.when(i == 0)
    def _():
        acc[...] = jnp.zeros_like(acc)

    acc[0, :] += jnp.sum(out, axis=0)
    acc[1, :] += jnp.sum(out * out, axis=0)
    stats_ref[...] = acc[...]


def _merge(acc, b, n):
    return pl.pallas_call(
        _merge_body,
        grid=(n // NBLK,),
        in_specs=[
            pl.BlockSpec((NBLK, DE), lambda i: (i, 0)),
            pl.BlockSpec((1, D), lambda i: (0, 0)),
        ],
        out_specs=[
            pl.BlockSpec((NBLK, D), lambda i: (i, 0)),
            pl.BlockSpec((2, D), lambda i: (0, 0)),
        ],
        out_shape=[
            jax.ShapeDtypeStruct((n, D), jnp.float32),
            jax.ShapeDtypeStruct((2, D), jnp.float32),
        ],
        scratch_shapes=[pltpu.VMEM((2, D), jnp.float32)],
    )(acc, b.reshape(1, D))


def _norm_body(x_ref, mustd_ref, o_ref):
    o_ref[...] = (x_ref[...] - mustd_ref[0, :]) * mustd_ref[1, :]


def _normalize(x, mustd):
    n = x.shape[0]
    return pl.pallas_call(
        _norm_body,
        grid=(n // NBLK,),
        in_specs=[
            pl.BlockSpec((NBLK, D), lambda i: (i, 0)),
            pl.BlockSpec((2, D), lambda i: (0, 0)),
        ],
        out_specs=pl.BlockSpec((NBLK, D), lambda i: (i, 0)),
        out_shape=jax.ShapeDtypeStruct((n, D), jnp.float32),
    )(x, mustd)


def _sc_edge_pass(h_ext, sd, src2, dst2, et2, n, nch):
    ngrp = nch // 4
    rsub = (n // NS) // 8 * 8
    rlast = n - (NS - 1) * rsub
    nz = rsub // 48

    mesh = plsc.VectorSubcoreMesh(
        core_axis_name="c", subcore_axis_name="s", num_cores=1)
    cp = pltpu.CompilerParams()
    for fld, val in (("needs_layout_passes", False),
                     ("use_tc_tiling_on_sc", False)):
        if fld in pltpu.CompilerParams.__dataclass_fields__:
            cp = dataclasses.replace(cp, **{fld: val})

    @functools.partial(
        pl.kernel,
        out_type=jax.ShapeDtypeStruct((n, DE), jnp.float32),
        mesh=mesh,
        compiler_params=cp,
        scratch_types=[
            pltpu.VMEM((2, n), jnp.float32),
            pltpu.VMEM((2, CHUNK), jnp.int32),
            pltpu.VMEM((2, CHUNK), jnp.int32),
            pltpu.VMEM((2, CHUNK), jnp.float32),
            pltpu.VMEM((2, CHUNK), jnp.int32),
            pltpu.VMEM((2, CHUNK), jnp.int32),
            pltpu.VMEM((2, CHUNK), jnp.float32),
            pltpu.VMEM((CHUNK, DE), jnp.float32),
            pltpu.VMEM((CHUNK, DE), jnp.float32),
            pltpu.SemaphoreType.DMA,
            pltpu.SemaphoreType.DMA,
            pltpu.SemaphoreType.DMA,
            pltpu.SemaphoreType.DMA,
            pltpu.SemaphoreType.DMA,
            pltpu.SemaphoreType.DMA,
            pltpu.VMEM_SHARED((n, DE), jnp.float32),
        ],
    )
    def k(h_hbm, sd_hbm, src_hbm, dst_hbm, et_hbm, out_hbm,
          sd_v, srcA, dstA, etA, srcB, dstB, etB, rowsA, rowsB,
          gA, gB, sA, sB, wA, wB, acc_sh):
        sid = lax.axis_index("s")

        zero16 = jnp.zeros((LN,), jnp.float32)

        @pl.loop(0, 48)
        def _(r):
            for cc in range(DE // LN):
                rowsA[r, pl.ds(cc * LN, LN)] = zero16

        pltpu.sync_copy(sd_hbm, sd_v)

        base = sid * rsub

        @pl.loop(0, nz)
        def _(j):
            pltpu.sync_copy(rowsA.at[pl.ds(0, 48)],
                            acc_sh.at[pl.ds(base + j * 48, 48)])

        @pl.when(sid == NS - 1)
        def _():
            pltpu.sync_copy(rowsA.at[pl.ds(0, 16)],
                            acc_sh.at[pl.ds(base + nz * 48, 16)])

        plsc.subcore_barrier()

        zeros_i = jnp.zeros((LN,), jnp.int32)
        ones_i = jnp.ones((LN,), jnp.int32)

        def compute(rows_v, srcw, dstw, etw, r):
            @pl.loop(0, CHUNK // LN)
            def _(g):
                isv = srcw[r, pl.ds(g * LN, LN)]
                idv = dstw[r, pl.ds(g * LN, LN)]
                sg = plsc.load_gather(sd_v, [zeros_i, isv])
                dg = plsc.load_gather(sd_v, [ones_i, idv])
                a = sg + dg + etw[r, pl.ds(g * LN, LN)]
                a = jnp.maximum(a, a * 0.2)
                exv = jnp.exp(a)
                for j in range(LN):
                    w = exv[j]
                    e = g * LN + j
                    for rr in range(DE // LN):
                        slc = (e, pl.ds(rr * LN, LN))
                        rows_v[slc] = rows_v[slc] * w

        def win_fetch(p, dst3, sem):
            sw, dw, ew = dst3
            pltpu.async_copy(src_hbm.at[sid, pl.ds(p * 2, 2)], sw, sem)
            pltpu.async_copy(dst_hbm.at[sid, pl.ds(p * 2, 2)], dw, sem)
            pltpu.async_copy(et_hbm.at[sid, pl.ds(p * 2, 2)], ew, sem)

        def win_wait(p, dst3, sem):
            sw, dw, ew = dst3
            pltpu.make_async_copy(src_hbm.at[sid, pl.ds(p * 2, 2)], sw, sem).wait()
            pltpu.make_async_copy(dst_hbm.at[sid, pl.ds(p * 2, 2)], dw, sem).wait()
            pltpu.make_async_copy(et_hbm.at[sid, pl.ds(p * 2, 2)], ew, sem).wait()

        def gather(srcw, r, rows_v, sem):
            pltpu.async_copy(h_hbm.at[srcw.at[r]], rows_v, sem)

        def gather_wait(srcw, r, rows_v, sem):
            pltpu.make_async_copy(h_hbm.at[srcw.at[r]], rows_v, sem).wait()

        def scatter(rows_v, dstw, r, sem):
            pltpu.async_copy(rows_v, acc_sh.at[dstw.at[r]], sem, add=True)

        def scatter_wait(rows_v, dstw, r, sem):
            pltpu.make_async_copy(rows_v, acc_sh.at[dstw.at[r]], sem).wait()

        winA = (srcA, dstA, etA)
        winB = (srcB, dstB, etB)

        win_fetch(0, winA, wA)
        win_wait(0, winA, wA)
        win_fetch(1, winB, wB)
        gather(srcA, 0, rowsA, gA)
        gather(srcA, 1, rowsB, gB)

        @pl.loop(0, ngrp - 1)
        def _(grp):
            gather_wait(srcA, 0, rowsA, gA)
            compute(rowsA, srcA, dstA, etA, 0)
            scatter(rowsA, dstA, 0, sA)
            gather_wait(srcA, 1, rowsB, gB)
            compute(rowsB, srcA, dstA, etA, 1)
            scatter(rowsB, dstA, 1, sB)
            win_wait(2 * grp + 1, winB, wB)
            scatter_wait(rowsA, dstA, 0, sA)
            gather(srcB, 0, rowsA, gA)
            scatter_wait(rowsB, dstA, 1, sB)
            gather(srcB, 1, rowsB, gB)
            win_fetch(2 * grp + 2, winA, wA)
            gather_wait(srcB, 0, rowsA, gA)
            compute(rowsA, srcB, dstB, etB, 0)
            scatter(rowsA, dstB, 0, sA)
            gather_wait(srcB, 1, rowsB, gB)
            compute(rowsB, srcB, dstB, etB, 1)
            scatter(rowsB, dstB, 1, sB)
            win_wait(2 * grp + 2, winA, wA)
            scatter_wait(rowsA, dstB, 0, sA)
            gather(srcA, 0, rowsA, gA)
            scatter_wait(rowsB, dstB, 1, sB)
            gather(srcA, 1, rowsB, gB)
            win_fetch(2 * grp + 3, winB, wB)

        gather_wait(srcA, 0, rowsA, gA)
        compute(rowsA, srcA, dstA, etA, 0)
        scatter(rowsA, dstA, 0, sA)
        gather_wait(srcA, 1, rowsB, gB)
        compute(rowsB, srcA, dstA, etA, 1)
        scatter(rowsB, dstA, 1, sB)
        win_wait(2 * ngrp - 1, winB, wB)
        scatter_wait(rowsA, dstA, 0, sA)
        gather(srcB, 0, rowsA, gA)
        scatter_wait(rowsB, dstA, 1, sB)
        gather(srcB, 1, rowsB, gB)
        gather_wait(srcB, 0, rowsA, gA)
        compute(rowsA, srcB, dstB, etB, 0)
        scatter(rowsA, dstB, 0, sA)
        gather_wait(srcB, 1, rowsB, gB)
        compute(rowsB, srcB, dstB, etB, 1)
        scatter(rowsB, dstB, 1, sB)
        scatter_wait(rowsA, dstB, 0, sA)
        scatter_wait(rowsB, dstB, 1, sB)

        plsc.subcore_barrier()

        pltpu.sync_copy(acc_sh.at[pl.ds(base, rsub)],
                        out_hbm.at[pl.ds(base, rsub)])

        @pl.when(sid == NS - 1)
        def _():
            pltpu.sync_copy(acc_sh.at[pl.ds(base + rsub, rlast - rsub)],
                            out_hbm.at[pl.ds(base + rsub, rlast - rsub)])

    return k(h_ext, sd, src2, dst2, et2)


def kernel(x, edge_index, edge_att, W1, We1, as1, ad1, ae1, b1,
           W2, We2, as2, ad2, ae2, b2):
    n = x.shape[0]
    e = edge_index.shape[1]
    ep_total = e + n
    nw = NS
    nch = _ceil_div(_ceil_div(ep_total, nw * CHUNK), 4) * 4
    ep = nw * CHUNK * nch
    pad = ep - ep_total

    loops = jnp.arange(n, dtype=jnp.int32)
    src = jnp.concatenate(
        [edge_index[0].astype(jnp.int32), loops, jnp.zeros((pad,), jnp.int32)])
    dst = jnp.concatenate(
        [edge_index[1].astype(jnp.int32), loops, jnp.zeros((pad,), jnp.int32)])
    src2 = src.reshape(nw, nch, CHUNK)
    dst2 = dst.reshape(nw, nch, CHUNK)

    ec1 = We1 @ ae1
    ec2 = We2 @ ae2
    et1_main, et1_sum = _edge_terms(edge_att, ec1)
    et2_main, et2_sum = _edge_terms(edge_att, ec2)

    def pack_et(et_main, et_sum):
        self_term = jnp.broadcast_to(jnp.sum(et_sum) / e, (n,))
        et = jnp.concatenate(
            [et_main[0], self_term, jnp.full((pad,), -1e30, jnp.float32)])
        return et.reshape(nw, nch, CHUNK)

    et1 = pack_et(et1_main, et1_sum)
    et2 = pack_et(et2_main, et2_sum)

    def bn_mustd(stats):
        mu = stats[0] / n
        var = stats[1] / n - mu * mu
        return jnp.stack([mu, 1.0 / jnp.sqrt(var + 1e-5)])

    h1e, sd1 = _project(x, W1, as1, ad1)
    sd1 = sd1.transpose(1, 0, 2).reshape(2, n)
    acc1 = _sc_edge_pass(h1e, sd1, src2, dst2, et1, n, nch)
    out1, stats1 = _merge(acc1, b1, n)

    h2e, sd2 = _project(out1, W2, as2, ad2, mustd=bn_mustd(stats1))
    sd2 = sd2.transpose(1, 0, 2).reshape(2, n)
    acc2 = _sc_edge_pass(h2e, sd2, src2, dst2, et2, n, nch)
    out2, stats2 = _merge(acc2, b2, n)

    return _normalize(out2, bn_mustd(stats2))
